# baseline (speedup 1.0000x reference)
import functools

import jax
import jax.numpy as jnp
from jax import lax
from jax.experimental import pallas as pl
from jax.experimental.pallas import tpu as pltpu
from jax.experimental.pallas import tpu_sc as plsc

B = 16
SRC = 2048
SECL = 16
WORDL = SRC // SECL
H = 512
D2 = 2 * H
EMB = 256
VOCAB = 50000
NOOV = 200
NV_TILE = 2048
NV = (VOCAB + NV_TILE - 1) // NV_TILE
SRC_TILE = 256
NS = SRC // SRC_TILE
SC_TILE = 128
NSC = SRC // SC_TILE
WPAD = 50432


def _prelude_compute(tok_ref, emb_hbm, pc_ref, h0_ref, c0_ref, sec2d_ref,
                     secmask_ref, Wc_ref, bc_ref, Wih_ref, Whh_ref, bih_ref,
                     bhh_ref, Wd_ref, bd_ref, Wf_ref, vsec_ref, Wdp_ref,
                     bdp_ref,
                     h1_ref, c1_ref, x_ref, dec_ref, sattn_ref, dpb_ref,
                     xemb, sem):
    for b in range(B):
        pltpu.make_async_copy(emb_hbm.at[tok_ref[b]], xemb.at[b], sem).start()
    for b in range(B):
        pltpu.make_async_copy(emb_hbm.at[tok_ref[b]], xemb.at[b], sem).wait()
    x_in = jnp.concatenate([pc_ref[...], xemb[...]], axis=1)
    x = lax.dot_general(x_in, Wc_ref[...], (((1,), (1,)), ((), ())),
                        preferred_element_type=jnp.float32) + bc_ref[...]
    gates = (lax.dot_general(x, Wih_ref[...], (((1,), (1,)), ((), ())),
                             preferred_element_type=jnp.float32)
             + bih_ref[...]
             + lax.dot_general(h0_ref[...], Whh_ref[...],
                               (((1,), (1,)), ((), ())),
                               preferred_element_type=jnp.float32)
             + bhh_ref[...])
    i_g = jax.nn.sigmoid(gates[:, 0 * H:1 * H])
    f_g = jax.nn.sigmoid(gates[:, 1 * H:2 * H])
    g_g = jnp.tanh(gates[:, 2 * H:3 * H])
    o_g = jax.nn.sigmoid(gates[:, 3 * H:4 * H])
    c1 = f_g * c0_ref[...] + i_g * g_g
    h1 = o_g * jnp.tanh(c1)
    dec = jnp.concatenate([h1, c1], axis=1)
    feat = lax.dot_general(sec2d_ref[...], Wf_ref[...], (((1,), (1,)), ((), ())),
                           preferred_element_type=jnp.float32)
    feat = feat.reshape(B, SECL, D2)
    dfeat = lax.dot_general(dec, Wd_ref[...], (((1,), (1,)), ((), ())),
                            preferred_element_type=jnp.float32) + bd_ref[...]
    st = jnp.tanh(feat + dfeat[:, None, :])
    sscore = jnp.sum(st * vsec_ref[...][None, :, :], axis=2)
    m = jnp.max(sscore, axis=1, keepdims=True)
    e = jnp.exp(sscore - m)
    sa = (e / jnp.sum(e, axis=1, keepdims=True)) * secmask_ref[...]
    sattn = sa / jnp.sum(sa, axis=1, keepdims=True)
    dpb = lax.dot_general(dec, Wdp_ref[...], (((1,), (1,)), ((), ())),
                          preferred_element_type=jnp.float32) + bdp_ref[...]
    h1_ref[...] = h1
    c1_ref[...] = c1
    x_ref[...] = x
    dec_ref[...] = dec
    sattn_ref[...] = sattn
    dpb_ref[...] = dpb


def _prelude_body(tok_ref, emb_hbm, pc_ref, h0_ref, c0_ref, sec2d_ref,
                  secmask_ref, Wc_ref, bc_ref, Wih_ref, Whh_ref, bih_ref,
                  bhh_ref, Wd_ref, bd_ref, Wf_ref, vsec_ref, Wdp_ref,
                  bdp_ref,
                  h1_ref, c1_ref, x_ref, dec_ref, sattn_ref, dpb_ref,
                  xemb, sem):
    _prelude_compute(tok_ref, emb_hbm, pc_ref, h0_ref, c0_ref, sec2d_ref,
                     secmask_ref, Wc_ref, bc_ref, Wih_ref, Whh_ref, bih_ref,
                     bhh_ref, Wd_ref, bd_ref, Wf_ref, vsec_ref, Wdp_ref,
                     bdp_ref, h1_ref, c1_ref, x_ref, dec_ref, sattn_ref,
                     dpb_ref, xemb, sem)


def _prelude(tok, emb_table, prev_context, h0, c0, sec2d, sec_mask,
             Wc, bc, Wih, Whh, bih, bhh, Wd, bd, Wf, vsec, Wdp, bdp):
    vspec = pl.BlockSpec(memory_space=pltpu.VMEM)
    return pl.pallas_call(
        _prelude_body,
        in_specs=[pl.BlockSpec(memory_space=pltpu.SMEM),
                  pl.BlockSpec(memory_space=pl.ANY)] + [vspec] * 17,
        out_specs=[vspec] * 6,
        out_shape=[jax.ShapeDtypeStruct((B, H), jnp.float32),
                   jax.ShapeDtypeStruct((B, H), jnp.float32),
                   jax.ShapeDtypeStruct((B, EMB), jnp.float32),
                   jax.ShapeDtypeStruct((B, D2), jnp.float32),
                   jax.ShapeDtypeStruct((B, SECL), jnp.float32),
                   jax.ShapeDtypeStruct((B, D2), jnp.float32)],
        scratch_shapes=[pltpu.VMEM((B, EMB), jnp.float32),
                        pltpu.SemaphoreType.DMA],
    )(tok, emb_table, prev_context, h0, c0, sec2d, sec_mask,
      Wc, bc, Wih, Whh, bih, bhh, Wd, bd, Wf, vsec, Wdp, bdp)


def _score_body(ef_ref, cov_ref, dpb_ref, wcov_ref, vattn_ref, score_ref):
    a = (ef_ref[...] + dpb_ref[...][:, None, :]
         + cov_ref[...][:, :, None] * wcov_ref[...][None, :, :])
    score_ref[...] = jnp.sum(jnp.tanh(a) * vattn_ref[...][None, :, :], axis=2)


def _score(enc_feature, coverage, dpb, wcov, vattn):
    return pl.pallas_call(
        _score_body,
        grid=(NS,),
        in_specs=[
            pl.BlockSpec((B, SRC_TILE, D2), lambda i: (0, i, 0)),
            pl.BlockSpec((B, SRC_TILE), lambda i: (0, i)),
            pl.BlockSpec((B, D2), lambda i: (0, 0)),
            pl.BlockSpec((1, D2), lambda i: (0, 0)),
            pl.BlockSpec((1, D2), lambda i: (0, 0)),
        ],
        out_specs=pl.BlockSpec((B, SRC_TILE), lambda i: (0, i)),
        out_shape=jax.ShapeDtypeStruct((B, SRC), jnp.float32),
        compiler_params=pltpu.CompilerParams(
            dimension_semantics=("arbitrary",)),
    )(enc_feature, coverage, dpb, wcov, vattn)


def _attn_ctx_body(score_ref, secexp_ref, mask_ref, eo_ref, cov_ref,
                   x_ref, dec_ref, wpg_ref, bpg_ref,
                   ad_ref, covn_ref, ctx_ref, pg_ref,
                   ad_s, ctx_s):
    i = pl.program_id(0)

    @pl.when(i == 0)
    def _():
        sc = score_ref[...]
        m = jnp.max(sc, axis=1, keepdims=True)
        e = jnp.exp(sc - m)
        ad0 = (e / jnp.sum(e, axis=1, keepdims=True)) * mask_ref[...]
        ad0 = ad0 / jnp.sum(ad0, axis=1, keepdims=True)
        ad0 = ad0 * secexp_ref[...]
        ad0 = ad0 / jnp.sum(ad0, axis=1, keepdims=True)
        ad_s[...] = ad0
        ad_ref[...] = ad0
        covn_ref[...] = cov_ref[...] + ad0
        ctx_s[...] = jnp.zeros_like(ctx_s)

    ad_t = ad_s[:, pl.ds(i * SRC_TILE, SRC_TILE)]
    ctx_s[...] += lax.dot_general(ad_t, eo_ref[...],
                                  (((1,), (1,)), ((0,), (0,))),
                                  preferred_element_type=jnp.float32)

    @pl.when(i == NS - 1)
    def _():
        ctx = ctx_s[...]
        ctx_ref[...] = ctx
        cat = jnp.concatenate([ctx, dec_ref[...], x_ref[...]], axis=1)
        pg = jax.nn.sigmoid(
            jnp.sum(cat * wpg_ref[...], axis=1, keepdims=True) + bpg_ref[...])
        pg_ref[...] = pg


def _attn_ctx(score, secexp, enc_mask, enc_outputs, coverage, x, dec,
              wpg, bpg):
    full = lambda s: pl.BlockSpec(s, lambda i: tuple(0 for _ in s))
    return pl.pallas_call(
        _attn_ctx_body,
        grid=(NS,),
        in_specs=[
            full((B, SRC)),
            full((B, SRC)),
            full((B, SRC)),
            pl.BlockSpec((B, SRC_TILE, D2), lambda i: (0, i, 0)),
            full((B, SRC)),
            full((B, EMB)),
            full((B, D2)),
            full((1, D2 + D2 + EMB)),
            full((1, 1)),
        ],
        out_specs=[full((B, SRC)), full((B, SRC)), full((B, D2)),
                   full((B, 1))],
        out_shape=[jax.ShapeDtypeStruct((B, SRC), jnp.float32),
                   jax.ShapeDtypeStruct((B, SRC), jnp.float32),
                   jax.ShapeDtypeStruct((B, D2), jnp.float32),
                   jax.ShapeDtypeStruct((B, 1), jnp.float32)],
        scratch_shapes=[pltpu.VMEM((B, SRC), jnp.float32),
                        pltpu.VMEM((B, D2), jnp.float32)],
        compiler_params=pltpu.CompilerParams(
            dimension_semantics=("arbitrary",)),
    )(score, secexp, enc_mask, enc_outputs, coverage, x, dec, wpg, bpg)


def _vocab_body(xo_ref, b_ref, pg_ref, w_ref, scat_hbm, fin_ref,
                l_s, m_s, s_s, scat_v, sem):
    i = pl.program_id(0)

    @pl.when(i == 0)
    def _():
        for b in range(B):
            pltpu.make_async_copy(scat_hbm.at[pl.ds(b * WPAD, WPAD)],
                                  scat_v.at[b], sem).start()

    l = lax.dot_general(xo_ref[...], w_ref[...], (((1,), (1,)), ((), ())),
                        preferred_element_type=jnp.float32) + b_ref[...]
    col = i * NV_TILE + lax.broadcasted_iota(jnp.int32, (B, NV_TILE), 1)
    l = jnp.where(col < VOCAB, l, -1e30)
    l_s[:, pl.ds(i * NV_TILE, NV_TILE)] = l
    mt = jnp.max(l, axis=1, keepdims=True)

    @pl.when(i == 0)
    def _():
        m_s[...] = mt
        s_s[...] = jnp.sum(jnp.exp(l - mt), axis=1, keepdims=True)

    @pl.when(i > 0)
    def _():
        m_old = m_s[...]
        m_new = jnp.maximum(m_old, mt)
        s_s[...] = (s_s[...] * jnp.exp(m_old - m_new)
                    + jnp.sum(jnp.exp(l - m_new), axis=1, keepdims=True))
        m_s[...] = m_new

    @pl.when(i == NV - 1)
    def _():
        for b in range(B):
            pltpu.make_async_copy(scat_hbm.at[pl.ds(b * WPAD, WPAD)],
                                  scat_v.at[b], sem).wait()
        pg = pg_ref[...]
        v = pg * jnp.exp(l_s[...] - m_s[...]) / s_s[...]
        fin_ref[...] = (v[:, :VOCAB + NOOV]
                        + (1.0 - pg) * scat_v[:, :VOCAB + NOOV])


def _vocab(xo, W_out, b_out, pg, scat_flat):
    return pl.pallas_call(
        _vocab_body,
        grid=(NV,),
        in_specs=[
            pl.BlockSpec((B, 3 * H), lambda i: (0, 0)),
            pl.BlockSpec((1, NV_TILE), lambda i: (0, i)),
            pl.BlockSpec((B, 1), lambda i: (0, 0)),
            pl.BlockSpec((NV_TILE, 3 * H), lambda i: (i, 0)),
            pl.BlockSpec(memory_space=pl.ANY),
        ],
        out_specs=pl.BlockSpec((B, VOCAB + NOOV), lambda i: (0, 0)),
        out_shape=jax.ShapeDtypeStruct((B, VOCAB + NOOV), jnp.float32),
        scratch_shapes=[pltpu.VMEM((B, NV * NV_TILE), jnp.float32),
                        pltpu.VMEM((B, 1), jnp.float32),
                        pltpu.VMEM((B, 1), jnp.float32),
                        pltpu.VMEM((B, WPAD), jnp.float32),
                        pltpu.SemaphoreType.DMA],
        compiler_params=pltpu.CompilerParams(
            dimension_semantics=("arbitrary",)),
    )(xo, b_out, pg, W_out, scat_flat)


def _sc_scatter_body(zeros_hbm, idx_hbm, adp_hbm, out_hbm, shared, idxv,
                     idxa, valv, rowbuf):
    c = lax.axis_index("c")
    s = lax.axis_index("s")

    @pl.when(s < 8)
    def _():
        pltpu.sync_copy(zeros_hbm.at[pl.ds((8 * c + s) * WPAD, WPAD)], rowbuf)
        pltpu.sync_copy(rowbuf, shared.at[pl.ds(s * WPAD, WPAD)])

    r = s // 2
    half = s % 2
    chunk = (8 * c + r) * 2 + half
    pltpu.sync_copy(idx_hbm.at[chunk], idxv)
    pltpu.sync_copy(adp_hbm.at[chunk], valv)
    off = r * WPAD
    for j in range(8):
        for k in range(8):
            idxa[j, pl.ds(k * 16, 16)] = idxv[j, pl.ds(k * 16, 16)] + off
    plsc.subcore_barrier()
    for j in range(8):
        pltpu.sync_copy(valv.at[j], shared.at[idxa.at[j]], add=True)
    plsc.subcore_barrier()

    @pl.when(s < 8)
    def _():
        pltpu.sync_copy(shared.at[pl.ds(s * WPAD, WPAD)], rowbuf)
        pltpu.sync_copy(rowbuf, out_hbm.at[pl.ds((8 * c + s) * WPAD, WPAD)])


def _sc_scatter(zeros_flat, idx3, adp3):
    mesh = plsc.VectorSubcoreMesh(core_axis_name="c", subcore_axis_name="s")
    fn = functools.partial(
        pl.kernel,
        mesh=mesh,
        out_type=jax.ShapeDtypeStruct((B * WPAD,), jnp.float32),
        scratch_types=[
            pltpu.VMEM_SHARED((8 * WPAD,), jnp.float32),
            pltpu.VMEM((8, 128), jnp.int32),
            pltpu.VMEM((8, 128), jnp.int32),
            pltpu.VMEM((8, 128), jnp.float32),
            pltpu.VMEM((WPAD,), jnp.float32),
        ],
    )(_sc_scatter_body)
    return fn(zeros_flat, idx3, adp3)


def kernel(inp_tok, h0, c0, enc_outputs, enc_feature, enc_sec_output,
           enc_mask, sec_mask, prev_context, zeros_oov, enc_input_oov,
           coverage, focus, emb_table, W_comb, b_comb, W_ih, W_hh, b_ih,
           b_hh, W_d, b_d, W_feat, v_sec, W_dp, b_dp, v_attn, w_cov, W_pg,
           b_pg, W_out, b_out):
    tok = inp_tok.astype(jnp.int32)
    sec2d = enc_sec_output.reshape(B * SECL, D2)
    h1, c1, x, dec, sattn, dpb = _prelude(
        tok, emb_table, prev_context, h0, c0, sec2d, sec_mask,
        W_comb, b_comb.reshape(1, EMB), W_ih, W_hh,
        b_ih.reshape(1, 4 * H), b_hh.reshape(1, 4 * H), W_d,
        b_d.reshape(1, D2), W_feat, v_sec.reshape(1, D2), W_dp,
        b_dp.reshape(1, D2))
    score = _score(enc_feature, coverage, dpb, w_cov.reshape(1, D2),
                   v_attn.reshape(1, D2))
    secexp = jnp.repeat(sattn, WORDL, axis=1)
    ad, covn, ctx, pg = _attn_ctx(
        score, secexp, enc_mask, enc_outputs, coverage, x, dec,
        W_pg.reshape(1, D2 + D2 + EMB), b_pg.reshape(1, 1))
    idx3 = enc_input_oov.astype(jnp.int32).reshape(2 * B, 8, 128)
    ad3 = ad.reshape(2 * B, 8, 128)
    scat = _sc_scatter(jnp.zeros((B * WPAD,), jnp.float32), idx3, ad3)
    xo = jnp.concatenate([h1, ctx], axis=1)
    final = _vocab(xo, W_out, b_out.reshape(1, VOCAB), pg, scat)
    return (final, h1, c1, ctx, ad, covn)


# speedup vs baseline: 1.0376x; 1.0376x over previous
import functools

import jax
import jax.numpy as jnp
from jax import lax
from jax.experimental import pallas as pl
from jax.experimental.pallas import tpu as pltpu
from jax.experimental.pallas import tpu_sc as plsc

B = 16
SRC = 2048
SECL = 16
WORDL = SRC // SECL
H = 512
D2 = 2 * H
EMB = 256
VOCAB = 50000
NOOV = 200
NV_TILE = 2048
NV = (VOCAB + NV_TILE - 1) // NV_TILE
SRC_TILE = 256
NS = SRC // SRC_TILE
SC_TILE = 128
NSC = SRC // SC_TILE
WPAD = 50432


def _prelude_compute(tok_ref, emb_hbm, pc_ref, h0_ref, c0_ref, sec2d_ref,
                     secmask_ref, Wc_ref, bc_ref, Wih_ref, Whh_ref, bih_ref,
                     bhh_ref, Wd_ref, bd_ref, Wf_ref, vsec_ref, Wdp_ref,
                     bdp_ref,
                     h1_ref, c1_ref, x_ref, dec_ref, sattn_ref, dpb_ref,
                     xemb, sem):
    for b in range(B):
        pltpu.make_async_copy(emb_hbm.at[tok_ref[b]], xemb.at[b], sem).start()
    for b in range(B):
        pltpu.make_async_copy(emb_hbm.at[tok_ref[b]], xemb.at[b], sem).wait()
    x_in = jnp.concatenate([pc_ref[...], xemb[...]], axis=1)
    x = lax.dot_general(x_in, Wc_ref[...], (((1,), (1,)), ((), ())),
                        preferred_element_type=jnp.float32) + bc_ref[...]
    gates = (lax.dot_general(x, Wih_ref[...], (((1,), (1,)), ((), ())),
                             preferred_element_type=jnp.float32)
             + bih_ref[...]
             + lax.dot_general(h0_ref[...], Whh_ref[...],
                               (((1,), (1,)), ((), ())),
                               preferred_element_type=jnp.float32)
             + bhh_ref[...])
    i_g = jax.nn.sigmoid(gates[:, 0 * H:1 * H])
    f_g = jax.nn.sigmoid(gates[:, 1 * H:2 * H])
    g_g = jnp.tanh(gates[:, 2 * H:3 * H])
    o_g = jax.nn.sigmoid(gates[:, 3 * H:4 * H])
    c1 = f_g * c0_ref[...] + i_g * g_g
    h1 = o_g * jnp.tanh(c1)
    dec = jnp.concatenate([h1, c1], axis=1)
    feat = lax.dot_general(sec2d_ref[...], Wf_ref[...], (((1,), (1,)), ((), ())),
                           preferred_element_type=jnp.float32)
    feat = feat.reshape(B, SECL, D2)
    dfeat = lax.dot_general(dec, Wd_ref[...], (((1,), (1,)), ((), ())),
                            preferred_element_type=jnp.float32) + bd_ref[...]
    st = jnp.tanh(feat + dfeat[:, None, :])
    sscore = jnp.sum(st * vsec_ref[...][None, :, :], axis=2)
    m = jnp.max(sscore, axis=1, keepdims=True)
    e = jnp.exp(sscore - m)
    sa = (e / jnp.sum(e, axis=1, keepdims=True)) * secmask_ref[...]
    sattn = sa / jnp.sum(sa, axis=1, keepdims=True)
    dpb = lax.dot_general(dec, Wdp_ref[...], (((1,), (1,)), ((), ())),
                          preferred_element_type=jnp.float32) + bdp_ref[...]
    h1_ref[...] = h1
    c1_ref[...] = c1
    x_ref[...] = x
    dec_ref[...] = dec
    sattn_ref[...] = sattn
    dpb_ref[...] = dpb


def _prelude_body(tok_ref, emb_hbm, pc_ref, h0_ref, c0_ref, sec2d_ref,
                  secmask_ref, Wc_ref, bc_ref, Wih_ref, Whh_ref, bih_ref,
                  bhh_ref, Wd_ref, bd_ref, Wf_ref, vsec_ref, Wdp_ref,
                  bdp_ref,
                  h1_ref, c1_ref, x_ref, dec_ref, sattn_ref, dpb_ref,
                  xemb, sem):
    _prelude_compute(tok_ref, emb_hbm, pc_ref, h0_ref, c0_ref, sec2d_ref,
                     secmask_ref, Wc_ref, bc_ref, Wih_ref, Whh_ref, bih_ref,
                     bhh_ref, Wd_ref, bd_ref, Wf_ref, vsec_ref, Wdp_ref,
                     bdp_ref, h1_ref, c1_ref, x_ref, dec_ref, sattn_ref,
                     dpb_ref, xemb, sem)


def _prelude(tok, emb_table, prev_context, h0, c0, sec2d, sec_mask,
             Wc, bc, Wih, Whh, bih, bhh, Wd, bd, Wf, vsec, Wdp, bdp):
    vspec = pl.BlockSpec(memory_space=pltpu.VMEM)
    return pl.pallas_call(
        _prelude_body,
        in_specs=[pl.BlockSpec(memory_space=pltpu.SMEM),
                  pl.BlockSpec(memory_space=pl.ANY)] + [vspec] * 17,
        out_specs=[vspec] * 6,
        out_shape=[jax.ShapeDtypeStruct((B, H), jnp.float32),
                   jax.ShapeDtypeStruct((B, H), jnp.float32),
                   jax.ShapeDtypeStruct((B, EMB), jnp.float32),
                   jax.ShapeDtypeStruct((B, D2), jnp.float32),
                   jax.ShapeDtypeStruct((B, SECL), jnp.float32),
                   jax.ShapeDtypeStruct((B, D2), jnp.float32)],
        scratch_shapes=[pltpu.VMEM((B, EMB), jnp.float32),
                        pltpu.SemaphoreType.DMA],
    )(tok, emb_table, prev_context, h0, c0, sec2d, sec_mask,
      Wc, bc, Wih, Whh, bih, bhh, Wd, bd, Wf, vsec, Wdp, bdp)


def _score_body(ef_ref, cov_ref, dpb_ref, wcov_ref, vattn_ref, secexp_ref,
                mask_ref, ad_ref, covn_ref, score_s):
    i = pl.program_id(0)
    cov_t = cov_ref[:, pl.ds(i * SRC_TILE, SRC_TILE)]
    a = (ef_ref[...] + dpb_ref[...][:, None, :]
         + cov_t[:, :, None] * wcov_ref[...][None, :, :])
    score_s[:, pl.ds(i * SRC_TILE, SRC_TILE)] = jnp.sum(
        jnp.tanh(a) * vattn_ref[...][None, :, :], axis=2)

    @pl.when(i == NS - 1)
    def _():
        sc = score_s[...]
        m = jnp.max(sc, axis=1, keepdims=True)
        e = jnp.exp(sc - m)
        ad0 = (e / jnp.sum(e, axis=1, keepdims=True)) * mask_ref[...]
        ad0 = ad0 / jnp.sum(ad0, axis=1, keepdims=True)
        ad0 = ad0 * secexp_ref[...]
        ad0 = ad0 / jnp.sum(ad0, axis=1, keepdims=True)
        ad_ref[...] = ad0
        covn_ref[...] = cov_ref[...] + ad0


def _score(enc_feature, coverage, dpb, wcov, vattn, secexp, enc_mask):
    full = lambda s: pl.BlockSpec(s, lambda i: tuple(0 for _ in s))
    return pl.pallas_call(
        _score_body,
        grid=(NS,),
        in_specs=[
            pl.BlockSpec((B, SRC_TILE, D2), lambda i: (0, i, 0)),
            full((B, SRC)),
            pl.BlockSpec((B, D2), lambda i: (0, 0)),
            pl.BlockSpec((1, D2), lambda i: (0, 0)),
            pl.BlockSpec((1, D2), lambda i: (0, 0)),
            full((B, SRC)),
            full((B, SRC)),
        ],
        out_specs=[full((B, SRC)), full((B, SRC))],
        out_shape=[jax.ShapeDtypeStruct((B, SRC), jnp.float32),
                   jax.ShapeDtypeStruct((B, SRC), jnp.float32)],
        scratch_shapes=[pltpu.VMEM((B, SRC), jnp.float32)],
        compiler_params=pltpu.CompilerParams(
            dimension_semantics=("arbitrary",)),
    )(enc_feature, coverage, dpb, wcov, vattn, secexp, enc_mask)


def _attn_ctx_body(ad_ref, eo_ref, x_ref, dec_ref, wpg_ref, bpg_ref,
                   ctx_ref, pg_ref, ctx_s):
    i = pl.program_id(0)

    @pl.when(i == 0)
    def _():
        ctx_s[...] = jnp.zeros_like(ctx_s)

    ad_t = ad_ref[:, pl.ds(i * SRC_TILE, SRC_TILE)]
    ctx_s[...] += lax.dot_general(ad_t, eo_ref[...],
                                  (((1,), (1,)), ((0,), (0,))),
                                  preferred_element_type=jnp.float32)

    @pl.when(i == NS - 1)
    def _():
        ctx = ctx_s[...]
        ctx_ref[...] = ctx
        cat = jnp.concatenate([ctx, dec_ref[...], x_ref[...]], axis=1)
        pg = jax.nn.sigmoid(
            jnp.sum(cat * wpg_ref[...], axis=1, keepdims=True) + bpg_ref[...])
        pg_ref[...] = pg


def _attn_ctx(ad, enc_outputs, x, dec, wpg, bpg):
    full = lambda s: pl.BlockSpec(s, lambda i: tuple(0 for _ in s))
    return pl.pallas_call(
        _attn_ctx_body,
        grid=(NS,),
        in_specs=[
            full((B, SRC)),
            pl.BlockSpec((B, SRC_TILE, D2), lambda i: (0, i, 0)),
            full((B, EMB)),
            full((B, D2)),
            full((1, D2 + D2 + EMB)),
            full((1, 1)),
        ],
        out_specs=[full((B, D2)), full((B, 1))],
        out_shape=[jax.ShapeDtypeStruct((B, D2), jnp.float32),
                   jax.ShapeDtypeStruct((B, 1), jnp.float32)],
        scratch_shapes=[pltpu.VMEM((B, D2), jnp.float32)],
        compiler_params=pltpu.CompilerParams(
            dimension_semantics=("arbitrary",)),
    )(ad, enc_outputs, x, dec, wpg, bpg)


def _vocab_body(xo_ref, b_ref, pg_ref, w_ref, scat_hbm, fin_ref,
                l_s, m_s, s_s, scat_v, sem):
    i = pl.program_id(0)

    @pl.when(i == 0)
    def _():
        for b in range(B):
            pltpu.make_async_copy(scat_hbm.at[pl.ds(b * WPAD, WPAD)],
                                  scat_v.at[b], sem).start()

    l = lax.dot_general(xo_ref[...], w_ref[...], (((1,), (1,)), ((), ())),
                        preferred_element_type=jnp.float32) + b_ref[...]
    col = i * NV_TILE + lax.broadcasted_iota(jnp.int32, (B, NV_TILE), 1)
    l = jnp.where(col < VOCAB, l, -1e30)
    l_s[:, pl.ds(i * NV_TILE, NV_TILE)] = l
    mt = jnp.max(l, axis=1, keepdims=True)

    @pl.when(i == 0)
    def _():
        m_s[...] = mt
        s_s[...] = jnp.sum(jnp.exp(l - mt), axis=1, keepdims=True)

    @pl.when(i > 0)
    def _():
        m_old = m_s[...]
        m_new = jnp.maximum(m_old, mt)
        s_s[...] = (s_s[...] * jnp.exp(m_old - m_new)
                    + jnp.sum(jnp.exp(l - m_new), axis=1, keepdims=True))
        m_s[...] = m_new

    @pl.when(i == NV - 1)
    def _():
        for b in range(B):
            pltpu.make_async_copy(scat_hbm.at[pl.ds(b * WPAD, WPAD)],
                                  scat_v.at[b], sem).wait()
        pg = pg_ref[...]
        v = pg * jnp.exp(l_s[...] - m_s[...]) / s_s[...]
        fin_ref[...] = (v[:, :VOCAB + NOOV]
                        + (1.0 - pg) * scat_v[:, :VOCAB + NOOV])


def _vocab(xo, W_out, b_out, pg, scat_flat):
    return pl.pallas_call(
        _vocab_body,
        grid=(NV,),
        in_specs=[
            pl.BlockSpec((B, 3 * H), lambda i: (0, 0)),
            pl.BlockSpec((1, NV_TILE), lambda i: (0, i)),
            pl.BlockSpec((B, 1), lambda i: (0, 0)),
            pl.BlockSpec((NV_TILE, 3 * H), lambda i: (i, 0)),
            pl.BlockSpec(memory_space=pl.ANY),
        ],
        out_specs=pl.BlockSpec((B, VOCAB + NOOV), lambda i: (0, 0)),
        out_shape=jax.ShapeDtypeStruct((B, VOCAB + NOOV), jnp.float32),
        scratch_shapes=[pltpu.VMEM((B, NV * NV_TILE), jnp.float32),
                        pltpu.VMEM((B, 1), jnp.float32),
                        pltpu.VMEM((B, 1), jnp.float32),
                        pltpu.VMEM((B, WPAD), jnp.float32),
                        pltpu.SemaphoreType.DMA],
        compiler_params=pltpu.CompilerParams(
            dimension_semantics=("arbitrary",)),
    )(xo, b_out, pg, W_out, scat_flat)


def _sc_scatter_body(zeros_hbm, idx_hbm, adp_hbm, out_hbm, shared, idxv,
                     idxa, valv, rowbuf):
    c = lax.axis_index("c")
    s = lax.axis_index("s")

    @pl.when(s < 8)
    def _():
        pltpu.sync_copy(zeros_hbm.at[pl.ds((8 * c + s) * WPAD, WPAD)], rowbuf)
        pltpu.sync_copy(rowbuf, shared.at[pl.ds(s * WPAD, WPAD)])

    r = s // 2
    half = s % 2
    chunk = (8 * c + r) * 2 + half
    pltpu.sync_copy(idx_hbm.at[chunk], idxv)
    pltpu.sync_copy(adp_hbm.at[chunk], valv)
    off = r * WPAD
    for j in range(8):
        for k in range(8):
            idxa[j, pl.ds(k * 16, 16)] = idxv[j, pl.ds(k * 16, 16)] + off
    plsc.subcore_barrier()
    for j in range(8):
        pltpu.sync_copy(valv.at[j], shared.at[idxa.at[j]], add=True)
    plsc.subcore_barrier()

    @pl.when(s < 8)
    def _():
        pltpu.sync_copy(shared.at[pl.ds(s * WPAD, WPAD)], rowbuf)
        pltpu.sync_copy(rowbuf, out_hbm.at[pl.ds((8 * c + s) * WPAD, WPAD)])


def _sc_scatter(zeros_flat, idx3, adp3):
    mesh = plsc.VectorSubcoreMesh(core_axis_name="c", subcore_axis_name="s")
    fn = functools.partial(
        pl.kernel,
        mesh=mesh,
        out_type=jax.ShapeDtypeStruct((B * WPAD,), jnp.float32),
        scratch_types=[
            pltpu.VMEM_SHARED((8 * WPAD,), jnp.float32),
            pltpu.VMEM((8, 128), jnp.int32),
            pltpu.VMEM((8, 128), jnp.int32),
            pltpu.VMEM((8, 128), jnp.float32),
            pltpu.VMEM((WPAD,), jnp.float32),
        ],
    )(_sc_scatter_body)
    return fn(zeros_flat, idx3, adp3)


def kernel(inp_tok, h0, c0, enc_outputs, enc_feature, enc_sec_output,
           enc_mask, sec_mask, prev_context, zeros_oov, enc_input_oov,
           coverage, focus, emb_table, W_comb, b_comb, W_ih, W_hh, b_ih,
           b_hh, W_d, b_d, W_feat, v_sec, W_dp, b_dp, v_attn, w_cov, W_pg,
           b_pg, W_out, b_out):
    tok = inp_tok.astype(jnp.int32)
    sec2d = enc_sec_output.reshape(B * SECL, D2)
    h1, c1, x, dec, sattn, dpb = _prelude(
        tok, emb_table, prev_context, h0, c0, sec2d, sec_mask,
        W_comb, b_comb.reshape(1, EMB), W_ih, W_hh,
        b_ih.reshape(1, 4 * H), b_hh.reshape(1, 4 * H), W_d,
        b_d.reshape(1, D2), W_feat, v_sec.reshape(1, D2), W_dp,
        b_dp.reshape(1, D2))
    secexp = jnp.repeat(sattn, WORDL, axis=1)
    ad, covn = _score(enc_feature, coverage, dpb, w_cov.reshape(1, D2),
                      v_attn.reshape(1, D2), secexp, enc_mask)
    idx3 = enc_input_oov.astype(jnp.int32).reshape(2 * B, 8, 128)
    ad3 = ad.reshape(2 * B, 8, 128)
    scat = _sc_scatter(jnp.zeros((B * WPAD,), jnp.float32), idx3, ad3)
    ctx, pg = _attn_ctx(ad, enc_outputs, x, dec,
                        W_pg.reshape(1, D2 + D2 + EMB), b_pg.reshape(1, 1))
    xo = jnp.concatenate([h1, ctx], axis=1)
    final = _vocab(xo, W_out, b_out.reshape(1, VOCAB), pg, scat)
    return (final, h1, c1, ctx, ad, covn)


# speedup vs baseline: 1.0417x; 1.0040x over previous
import functools

import jax
import jax.numpy as jnp
from jax import lax
from jax.experimental import pallas as pl
from jax.experimental.pallas import tpu as pltpu
from jax.experimental.pallas import tpu_sc as plsc

B = 16
SRC = 2048
SECL = 16
WORDL = SRC // SECL
H = 512
D2 = 2 * H
EMB = 256
VOCAB = 50000
NOOV = 200
NV_TILE = 2048
NV = (VOCAB + NV_TILE - 1) // NV_TILE
SRC_TILE = 256
NS = SRC // SRC_TILE
SC_TILE = 128
NSC = SRC // SC_TILE
WPAD = 50432


def _prelude_compute(tok_ref, emb_hbm, pc_ref, h0_ref, c0_ref, sec2d_ref,
                     secmask_ref, Wc_ref, bc_ref, Wih_ref, Whh_ref, bih_ref,
                     bhh_ref, Wd_ref, bd_ref, Wf_ref, vsec_ref, Wdp_ref,
                     bdp_ref,
                     h1_ref, c1_ref, x_ref, dec_ref, sattn_ref, dpb_ref,
                     xemb, sem):
    for b in range(B):
        pltpu.make_async_copy(emb_hbm.at[tok_ref[b]], xemb.at[b], sem).start()
    for b in range(B):
        pltpu.make_async_copy(emb_hbm.at[tok_ref[b]], xemb.at[b], sem).wait()
    x_in = jnp.concatenate([pc_ref[...], xemb[...]], axis=1)
    x = lax.dot_general(x_in, Wc_ref[...], (((1,), (1,)), ((), ())),
                        preferred_element_type=jnp.float32) + bc_ref[...]
    gates = (lax.dot_general(x, Wih_ref[...], (((1,), (1,)), ((), ())),
                             preferred_element_type=jnp.float32)
             + bih_ref[...]
             + lax.dot_general(h0_ref[...], Whh_ref[...],
                               (((1,), (1,)), ((), ())),
                               preferred_element_type=jnp.float32)
             + bhh_ref[...])
    i_g = jax.nn.sigmoid(gates[:, 0 * H:1 * H])
    f_g = jax.nn.sigmoid(gates[:, 1 * H:2 * H])
    g_g = jnp.tanh(gates[:, 2 * H:3 * H])
    o_g = jax.nn.sigmoid(gates[:, 3 * H:4 * H])
    c1 = f_g * c0_ref[...] + i_g * g_g
    h1 = o_g * jnp.tanh(c1)
    dec = jnp.concatenate([h1, c1], axis=1)
    feat = lax.dot_general(sec2d_ref[...], Wf_ref[...], (((1,), (1,)), ((), ())),
                           preferred_element_type=jnp.float32)
    feat = feat.reshape(B, SECL, D2)
    dfeat = lax.dot_general(dec, Wd_ref[...], (((1,), (1,)), ((), ())),
                            preferred_element_type=jnp.float32) + bd_ref[...]
    st = jnp.tanh(feat + dfeat[:, None, :])
    sscore = jnp.sum(st * vsec_ref[...][None, :, :], axis=2)
    m = jnp.max(sscore, axis=1, keepdims=True)
    e = jnp.exp(sscore - m)
    sa = (e / jnp.sum(e, axis=1, keepdims=True)) * secmask_ref[...]
    sattn = sa / jnp.sum(sa, axis=1, keepdims=True)
    dpb = lax.dot_general(dec, Wdp_ref[...], (((1,), (1,)), ((), ())),
                          preferred_element_type=jnp.float32) + bdp_ref[...]
    h1_ref[...] = h1
    c1_ref[...] = c1
    x_ref[...] = x
    dec_ref[...] = dec
    sattn_ref[...] = sattn
    dpb_ref[...] = dpb


def _prelude_body(tok_ref, emb_hbm, pc_ref, h0_ref, c0_ref, sec2d_ref,
                  secmask_ref, Wc_ref, bc_ref, Wih_ref, Whh_ref, bih_ref,
                  bhh_ref, Wd_ref, bd_ref, Wf_ref, vsec_ref, Wdp_ref,
                  bdp_ref,
                  h1_ref, c1_ref, x_ref, dec_ref, sattn_ref, dpb_ref,
                  xemb, sem):
    _prelude_compute(tok_ref, emb_hbm, pc_ref, h0_ref, c0_ref, sec2d_ref,
                     secmask_ref, Wc_ref, bc_ref, Wih_ref, Whh_ref, bih_ref,
                     bhh_ref, Wd_ref, bd_ref, Wf_ref, vsec_ref, Wdp_ref,
                     bdp_ref, h1_ref, c1_ref, x_ref, dec_ref, sattn_ref,
                     dpb_ref, xemb, sem)


def _prelude(tok, emb_table, prev_context, h0, c0, sec2d, sec_mask,
             Wc, bc, Wih, Whh, bih, bhh, Wd, bd, Wf, vsec, Wdp, bdp):
    vspec = pl.BlockSpec(memory_space=pltpu.VMEM)
    return pl.pallas_call(
        _prelude_body,
        in_specs=[pl.BlockSpec(memory_space=pltpu.SMEM),
                  pl.BlockSpec(memory_space=pl.ANY)] + [vspec] * 17,
        out_specs=[vspec] * 6,
        out_shape=[jax.ShapeDtypeStruct((B, H), jnp.float32),
                   jax.ShapeDtypeStruct((B, H), jnp.float32),
                   jax.ShapeDtypeStruct((B, EMB), jnp.float32),
                   jax.ShapeDtypeStruct((B, D2), jnp.float32),
                   jax.ShapeDtypeStruct((B, SECL), jnp.float32),
                   jax.ShapeDtypeStruct((B, D2), jnp.float32)],
        scratch_shapes=[pltpu.VMEM((B, EMB), jnp.float32),
                        pltpu.SemaphoreType.DMA],
    )(tok, emb_table, prev_context, h0, c0, sec2d, sec_mask,
      Wc, bc, Wih, Whh, bih, bhh, Wd, bd, Wf, vsec, Wdp, bdp)


def _score_body(ef_ref, cov_ref, dpb_ref, wcov_ref, vattn_ref, secexp_ref,
                mask_ref, ad_ref, covn_ref, score_s):
    i = pl.program_id(0)
    cov_t = cov_ref[:, pl.ds(i * SRC_TILE, SRC_TILE)]
    a = (ef_ref[...] + dpb_ref[...][:, None, :]
         + cov_t[:, :, None] * wcov_ref[...][None, :, :])
    score_s[:, pl.ds(i * SRC_TILE, SRC_TILE)] = jnp.sum(
        jnp.tanh(a) * vattn_ref[...][None, :, :], axis=2)

    @pl.when(i == NS - 1)
    def _():
        sc = score_s[...]
        m = jnp.max(sc, axis=1, keepdims=True)
        e = jnp.exp(sc - m)
        ad0 = (e / jnp.sum(e, axis=1, keepdims=True)) * mask_ref[...]
        ad0 = ad0 / jnp.sum(ad0, axis=1, keepdims=True)
        ad0 = ad0 * secexp_ref[...]
        ad0 = ad0 / jnp.sum(ad0, axis=1, keepdims=True)
        ad_ref[...] = ad0
        covn_ref[...] = cov_ref[...] + ad0


def _score(enc_feature, coverage, dpb, wcov, vattn, secexp, enc_mask):
    full = lambda s: pl.BlockSpec(s, lambda i: tuple(0 for _ in s))
    return pl.pallas_call(
        _score_body,
        grid=(NS,),
        in_specs=[
            pl.BlockSpec((B, SRC_TILE, D2), lambda i: (0, i, 0)),
            full((B, SRC)),
            pl.BlockSpec((B, D2), lambda i: (0, 0)),
            pl.BlockSpec((1, D2), lambda i: (0, 0)),
            pl.BlockSpec((1, D2), lambda i: (0, 0)),
            full((B, SRC)),
            full((B, SRC)),
        ],
        out_specs=[full((B, SRC)), full((B, SRC))],
        out_shape=[jax.ShapeDtypeStruct((B, SRC), jnp.float32),
                   jax.ShapeDtypeStruct((B, SRC), jnp.float32)],
        scratch_shapes=[pltpu.VMEM((B, SRC), jnp.float32)],
        compiler_params=pltpu.CompilerParams(
            dimension_semantics=("arbitrary",)),
    )(enc_feature, coverage, dpb, wcov, vattn, secexp, enc_mask)


def _attn_ctx_body(ad_ref, eo_ref, x_ref, dec_ref, wpg_ref, bpg_ref,
                   ctx_ref, pg_ref, ctx_s):
    i = pl.program_id(0)

    @pl.when(i == 0)
    def _():
        ctx_s[...] = jnp.zeros_like(ctx_s)

    ad_t = ad_ref[:, pl.ds(i * SRC_TILE, SRC_TILE)]
    ctx_s[...] += lax.dot_general(ad_t, eo_ref[...],
                                  (((1,), (1,)), ((0,), (0,))),
                                  preferred_element_type=jnp.float32)

    @pl.when(i == NS - 1)
    def _():
        ctx = ctx_s[...]
        ctx_ref[...] = ctx
        cat = jnp.concatenate([ctx, dec_ref[...], x_ref[...]], axis=1)
        pg = jax.nn.sigmoid(
            jnp.sum(cat * wpg_ref[...], axis=1, keepdims=True) + bpg_ref[...])
        pg_ref[...] = pg


def _attn_ctx(ad, enc_outputs, x, dec, wpg, bpg):
    full = lambda s: pl.BlockSpec(s, lambda i: tuple(0 for _ in s))
    return pl.pallas_call(
        _attn_ctx_body,
        grid=(NS,),
        in_specs=[
            full((B, SRC)),
            pl.BlockSpec((B, SRC_TILE, D2), lambda i: (0, i, 0)),
            full((B, EMB)),
            full((B, D2)),
            full((1, D2 + D2 + EMB)),
            full((1, 1)),
        ],
        out_specs=[full((B, D2)), full((B, 1))],
        out_shape=[jax.ShapeDtypeStruct((B, D2), jnp.float32),
                   jax.ShapeDtypeStruct((B, 1), jnp.float32)],
        scratch_shapes=[pltpu.VMEM((B, D2), jnp.float32)],
        compiler_params=pltpu.CompilerParams(
            dimension_semantics=("arbitrary",)),
    )(ad, enc_outputs, x, dec, wpg, bpg)


def _vocab_body(xo_ref, b_ref, pg_ref, w_ref, scat_hbm, fin_ref,
                l_s, m_s, s_s, scat_v, sem):
    i = pl.program_id(0)

    @pl.when(i == 0)
    def _():
        for b in range(B):
            pltpu.make_async_copy(scat_hbm.at[pl.ds(b * WPAD, WPAD)],
                                  scat_v.at[b], sem).start()

    l = lax.dot_general(xo_ref[...], w_ref[...], (((1,), (1,)), ((), ())),
                        preferred_element_type=jnp.float32) + b_ref[...]
    col = i * NV_TILE + lax.broadcasted_iota(jnp.int32, (B, NV_TILE), 1)
    l = jnp.where(col < VOCAB, l, -1e30)
    l_s[:, pl.ds(i * NV_TILE, NV_TILE)] = l
    mt = jnp.max(l, axis=1, keepdims=True)

    @pl.when(i == 0)
    def _():
        m_s[...] = mt
        s_s[...] = jnp.sum(jnp.exp(l - mt), axis=1, keepdims=True)

    @pl.when(i > 0)
    def _():
        m_old = m_s[...]
        m_new = jnp.maximum(m_old, mt)
        s_s[...] = (s_s[...] * jnp.exp(m_old - m_new)
                    + jnp.sum(jnp.exp(l - m_new), axis=1, keepdims=True))
        m_s[...] = m_new

    @pl.when(i == NV - 1)
    def _():
        for b in range(B):
            pltpu.make_async_copy(scat_hbm.at[pl.ds(b * WPAD, WPAD)],
                                  scat_v.at[b], sem).wait()
        pg = pg_ref[...]
        v = pg * jnp.exp(l_s[...] - m_s[...]) / s_s[...]
        fin_ref[...] = (v[:, :VOCAB + NOOV]
                        + (1.0 - pg) * scat_v[:, :VOCAB + NOOV])


def _vocab(xo, W_out, b_out, pg, scat_flat):
    return pl.pallas_call(
        _vocab_body,
        grid=(NV,),
        in_specs=[
            pl.BlockSpec((B, 3 * H), lambda i: (0, 0)),
            pl.BlockSpec((1, NV_TILE), lambda i: (0, i)),
            pl.BlockSpec((B, 1), lambda i: (0, 0)),
            pl.BlockSpec((NV_TILE, 3 * H), lambda i: (i, 0)),
            pl.BlockSpec(memory_space=pl.ANY),
        ],
        out_specs=pl.BlockSpec((B, VOCAB + NOOV), lambda i: (0, 0)),
        out_shape=jax.ShapeDtypeStruct((B, VOCAB + NOOV), jnp.float32),
        scratch_shapes=[pltpu.VMEM((B, NV * NV_TILE), jnp.float32),
                        pltpu.VMEM((B, 1), jnp.float32),
                        pltpu.VMEM((B, 1), jnp.float32),
                        pltpu.VMEM((B, WPAD), jnp.float32),
                        pltpu.SemaphoreType.DMA],
        compiler_params=pltpu.CompilerParams(
            dimension_semantics=("arbitrary",)),
    )(xo, b_out, pg, W_out, scat_flat)


def _sc_scatter_body(idx_hbm, adp_hbm, out_hbm, shared, idxv,
                     idxa, valv, rowbuf):
    c = lax.axis_index("c")
    s = lax.axis_index("s")

    @pl.when(s < 8)
    def _():
        def _zero(k, carry):
            rowbuf[pl.ds(k * 16, 16)] = jnp.zeros((16,), jnp.float32)
            return carry
        lax.fori_loop(0, WPAD // 16, _zero, 0)
        pltpu.sync_copy(rowbuf, shared.at[pl.ds(s * WPAD, WPAD)])

    r = s // 2
    half = s % 2
    chunk = (8 * c + r) * 2 + half
    pltpu.sync_copy(idx_hbm.at[chunk], idxv)
    pltpu.sync_copy(adp_hbm.at[chunk], valv)
    off = r * WPAD
    for j in range(8):
        for k in range(8):
            idxa[j, pl.ds(k * 16, 16)] = idxv[j, pl.ds(k * 16, 16)] + off
    plsc.subcore_barrier()
    for j in range(8):
        pltpu.sync_copy(valv.at[j], shared.at[idxa.at[j]], add=True)
    plsc.subcore_barrier()

    @pl.when(s < 8)
    def _():
        pltpu.sync_copy(shared.at[pl.ds(s * WPAD, WPAD)], rowbuf)
        pltpu.sync_copy(rowbuf, out_hbm.at[pl.ds((8 * c + s) * WPAD, WPAD)])


def _sc_scatter(idx3, adp3):
    mesh = plsc.VectorSubcoreMesh(core_axis_name="c", subcore_axis_name="s")
    fn = functools.partial(
        pl.kernel,
        mesh=mesh,
        out_type=jax.ShapeDtypeStruct((B * WPAD,), jnp.float32),
        scratch_types=[
            pltpu.VMEM_SHARED((8 * WPAD,), jnp.float32),
            pltpu.VMEM((8, 128), jnp.int32),
            pltpu.VMEM((8, 128), jnp.int32),
            pltpu.VMEM((8, 128), jnp.float32),
            pltpu.VMEM((WPAD,), jnp.float32),
        ],
    )(_sc_scatter_body)
    return fn(idx3, adp3)


def kernel(inp_tok, h0, c0, enc_outputs, enc_feature, enc_sec_output,
           enc_mask, sec_mask, prev_context, zeros_oov, enc_input_oov,
           coverage, focus, emb_table, W_comb, b_comb, W_ih, W_hh, b_ih,
           b_hh, W_d, b_d, W_feat, v_sec, W_dp, b_dp, v_attn, w_cov, W_pg,
           b_pg, W_out, b_out):
    tok = inp_tok.astype(jnp.int32)
    sec2d = enc_sec_output.reshape(B * SECL, D2)
    h1, c1, x, dec, sattn, dpb = _prelude(
        tok, emb_table, prev_context, h0, c0, sec2d, sec_mask,
        W_comb, b_comb.reshape(1, EMB), W_ih, W_hh,
        b_ih.reshape(1, 4 * H), b_hh.reshape(1, 4 * H), W_d,
        b_d.reshape(1, D2), W_feat, v_sec.reshape(1, D2), W_dp,
        b_dp.reshape(1, D2))
    secexp = jnp.repeat(sattn, WORDL, axis=1)
    ad, covn = _score(enc_feature, coverage, dpb, w_cov.reshape(1, D2),
                      v_attn.reshape(1, D2), secexp, enc_mask)
    idx3 = enc_input_oov.astype(jnp.int32).reshape(2 * B, 8, 128)
    ad3 = ad.reshape(2 * B, 8, 128)
    scat = _sc_scatter(idx3, ad3)
    ctx, pg = _attn_ctx(ad, enc_outputs, x, dec,
                        W_pg.reshape(1, D2 + D2 + EMB), b_pg.reshape(1, 1))
    xo = jnp.concatenate([h1, ctx], axis=1)
    final = _vocab(xo, W_out, b_out.reshape(1, VOCAB), pg, scat)
    return (final, h1, c1, ctx, ad, covn)
